# Initial kernel scaffold; baseline (speedup 1.0000x reference)
import functools

import jax
import jax.numpy as jnp
from jax import lax
from jax.experimental import pallas as pl
from jax.experimental.pallas import tpu as pltpu
from jax.experimental.pallas import tpu_sc as plsc

N, K, D, O = 10000, 32, 128, 128

NC, NS = 2, 16
NW = NC * NS
NPAD = 10240
RPW = NPAD // NW
CH = 4
NCHUNK = RPW // CH

MM_BM = 200


def _mm_body(h_ref, wt_ref, y_ref):
    y_ref[...] = jnp.dot(h_ref[...], wt_ref[...],
                         preferred_element_type=jnp.float32)


def _matmul(h, wt):
    return pl.pallas_call(
        _mm_body,
        grid=(N // MM_BM,),
        in_specs=[
            pl.BlockSpec((MM_BM, D), lambda i: (i, 0)),
            pl.BlockSpec((D, K * O), lambda i: (0, 0)),
        ],
        out_specs=pl.BlockSpec((MM_BM, K * O), lambda i: (i, 0)),
        out_shape=jax.ShapeDtypeStruct((N, K * O), jnp.float32),
    )(h, wt)


def _newton_rsqrt(sq):
    bits = plsc.bitcast(sq, jnp.int32)
    seed = jnp.int32(0x5F3759DF) - lax.shift_right_logical(bits, 1)
    y = plsc.bitcast(seed, jnp.float32)
    for _ in range(3):
        y = y * (jnp.float32(1.5) - jnp.float32(0.5) * sq * y * y)
    return jnp.where(sq == jnp.float32(0.0), jnp.float32(2.0), y)


def _sc_reduce(y2, posp, idxp):
    mesh = plsc.VectorSubcoreMesh(core_axis_name="c", subcore_axis_name="s")

    @functools.partial(
        pl.kernel,
        out_type=jax.ShapeDtypeStruct((NPAD * O,), jnp.float32),
        mesh=mesh,
        scratch_types=[
            pltpu.VMEM((NPAD,), jnp.float32),
            pltpu.VMEM((NPAD,), jnp.float32),
            pltpu.VMEM((NPAD,), jnp.float32),
            pltpu.VMEM((RPW * K,), jnp.int32),
            pltpu.VMEM((CH * K,), jnp.int32),
            pltpu.VMEM((CH * K, O), jnp.float32),
            pltpu.VMEM((CH * K,), jnp.float32),
            pltpu.VMEM((CH * O,), jnp.float32),
            pltpu.SemaphoreType.DMA,
        ],
    )
    def sc_kernel(y_hbm, pos_hbm, idx_hbm, out_hbm,
                  posx, posy, posz, idxv, gidx, rows, inv, outb, sem):
        wid = lax.axis_index("s") * NC + lax.axis_index("c")
        base = wid * RPW
        pltpu.sync_copy(pos_hbm.at[0], posx)
        pltpu.sync_copy(pos_hbm.at[1], posy)
        pltpu.sync_copy(pos_hbm.at[2], posz)
        pltpu.sync_copy(idx_hbm.at[pl.ds(base * K, RPW * K)], idxv)
        lane = lax.iota(jnp.int32, 16)

        def chunk(c, carry):
            for ii in range(CH):
                r = c * CH + ii
                g = base + r
                xi, yi, zi = posx[g], posy[g], posz[g]
                for half in range(2):
                    v = idxv[pl.ds(r * K + half * 16, 16)]
                    gidx[pl.ds(ii * K + half * 16, 16)] = (
                        v * K + (lane + half * 16))
                    dx = xi - plsc.load_gather(posx, [v])
                    dy = yi - plsc.load_gather(posy, [v])
                    dz = zi - plsc.load_gather(posz, [v])
                    sq = dx * dx + dy * dy + dz * dz
                    inv[pl.ds(ii * K + half * 16, 16)] = _newton_rsqrt(sq)
            pltpu.async_copy(y_hbm.at[gidx], rows, sem).wait()
            for ii in range(CH):
                def acc_body(kk, acc):
                    j = ii * K + kk
                    s = inv[j]
                    return tuple(acc[u] + s * rows[j, pl.ds(u * 16, 16)]
                                 for u in range(8))
                acc = lax.fori_loop(
                    0, K, acc_body,
                    tuple(jnp.zeros((16,), jnp.float32) for _ in range(8)))
                for u in range(8):
                    outb[pl.ds(ii * O + u * 16, 16)] = acc[u]
            pltpu.sync_copy(
                outb, out_hbm.at[pl.ds((base + c * CH) * O, CH * O)])
            return carry

        lax.fori_loop(0, NCHUNK, chunk, 0)

    return sc_kernel(y2, posp, idxp)


@jax.jit
def kernel(h, pos, neighbor_idx, W):
    wt = W.reshape(K, D, O).transpose(1, 0, 2).reshape(D, K * O)
    y2 = _matmul(h, wt).reshape(N * K, O)
    posp = jnp.zeros((3, NPAD), jnp.float32).at[:, :N].set(pos.T)
    idxp = (jnp.zeros((NPAD, K), jnp.int32)
            .at[:N].set(neighbor_idx).reshape(NPAD * K))
    out = _sc_reduce(y2, posp, idxp)
    return out.reshape(NPAD, O)[:N]


# speedup vs baseline: 3.5025x; 3.5025x over previous
import functools

import jax
import jax.numpy as jnp
from jax import lax
from jax.experimental import pallas as pl
from jax.experimental.pallas import tpu as pltpu
from jax.experimental.pallas import tpu_sc as plsc

N, K, D, O = 10000, 32, 128, 128

NC, NS = 2, 16
NW = NC * NS
NPAD = 10240
RPW = NPAD // NW
CH = 4
NCHUNK = RPW // CH

MM_BM = 200


def _mm_body(h_ref, wt_ref, y_ref):
    y_ref[...] = jnp.dot(h_ref[...], wt_ref[...],
                         preferred_element_type=jnp.float32)


def _matmul(h, wt):
    return pl.pallas_call(
        _mm_body,
        grid=(N // MM_BM,),
        in_specs=[
            pl.BlockSpec((MM_BM, D), lambda i: (i, 0)),
            pl.BlockSpec((D, K * O), lambda i: (0, 0)),
        ],
        out_specs=pl.BlockSpec((MM_BM, K * O), lambda i: (i, 0)),
        out_shape=jax.ShapeDtypeStruct((N, K * O), jnp.float32),
    )(h, wt)


def _newton_rsqrt(sq):
    bits = plsc.bitcast(sq, jnp.int32)
    seed = jnp.int32(0x5F3759DF) - lax.shift_right_logical(bits, 1)
    y = plsc.bitcast(seed, jnp.float32)
    for _ in range(3):
        y = y * (jnp.float32(1.5) - jnp.float32(0.5) * sq * y * y)
    return jnp.where(sq == jnp.float32(0.0), jnp.float32(2.0), y)


def _sc_reduce(y2, posp, idxp):
    mesh = plsc.VectorSubcoreMesh(core_axis_name="c", subcore_axis_name="s")

    @functools.partial(
        pl.kernel,
        out_type=jax.ShapeDtypeStruct((NPAD * O,), jnp.float32),
        mesh=mesh,
        scratch_types=[
            pltpu.VMEM((NPAD + 16,), jnp.float32),
            pltpu.VMEM((NPAD + 16,), jnp.float32),
            pltpu.VMEM((NPAD + 16,), jnp.float32),
            pltpu.VMEM((RPW * K,), jnp.int32),
            pltpu.VMEM((CH * K,), jnp.int32),
            pltpu.VMEM((CH * K, O), jnp.float32),
            pltpu.VMEM((CH * K + 16,), jnp.float32),
            pltpu.VMEM((CH * O,), jnp.float32),
            pltpu.SemaphoreType.DMA,
        ],
        compiler_params=pltpu.CompilerParams(needs_layout_passes=False),
    )
    def sc_kernel(y_hbm, pos_hbm, idx_hbm, out_hbm,
                  posx, posy, posz, idxv, gidx, rows, inv, outb, sem):
        wid = lax.axis_index("s") * NC + lax.axis_index("c")
        base = wid * RPW
        pltpu.sync_copy(pos_hbm.at[pl.ds(0, NPAD)], posx.at[pl.ds(0, NPAD)])
        pltpu.sync_copy(pos_hbm.at[pl.ds(NPAD, NPAD)],
                        posy.at[pl.ds(0, NPAD)])
        pltpu.sync_copy(pos_hbm.at[pl.ds(2 * NPAD, NPAD)],
                        posz.at[pl.ds(0, NPAD)])
        pltpu.sync_copy(idx_hbm.at[pl.ds(base * K, RPW * K)], idxv)
        lane = lax.iota(jnp.int32, 16)

        def chunk(c, carry):
            for ii in range(CH):
                r = c * CH + ii
                g = base + r
                xi = posx[pl.ds(g, 16)][0]
                yi = posy[pl.ds(g, 16)][0]
                zi = posz[pl.ds(g, 16)][0]
                for half in range(2):
                    v = idxv[pl.ds(r * K + half * 16, 16)]
                    gidx[pl.ds(ii * K + half * 16, 16)] = (
                        v * K + (lane + half * 16))
                    dx = xi - plsc.load_gather(posx, [v])
                    dy = yi - plsc.load_gather(posy, [v])
                    dz = zi - plsc.load_gather(posz, [v])
                    sq = dx * dx + dy * dy + dz * dz
                    inv[pl.ds(ii * K + half * 16, 16)] = _newton_rsqrt(sq)
            pltpu.async_copy(y_hbm.at[gidx], rows, sem).wait()
            for ii in range(CH):
                def acc_body(kk, acc):
                    j = ii * K + kk
                    s = inv[pl.ds(j, 16)][0]
                    return tuple(acc[u] + s * rows[j, pl.ds(u * 16, 16)]
                                 for u in range(8))
                acc = lax.fori_loop(
                    0, K, acc_body,
                    tuple(jnp.zeros((16,), jnp.float32) for _ in range(8)))
                for u in range(8):
                    outb[pl.ds(ii * O + u * 16, 16)] = acc[u]
            pltpu.sync_copy(
                outb, out_hbm.at[pl.ds((base + c * CH) * O, CH * O)])
            return carry

        lax.fori_loop(0, NCHUNK, chunk, 0)

    return sc_kernel(y2, posp, idxp)


@jax.jit
def kernel(h, pos, neighbor_idx, W):
    wt = W.reshape(K, D, O).transpose(1, 0, 2).reshape(D, K * O)
    y2 = _matmul(h, wt).reshape(N * K, O)
    posp = (jnp.zeros((3, NPAD), jnp.float32)
            .at[:, :N].set(pos.T).reshape(3 * NPAD))
    idxp = (jnp.zeros((NPAD, K), jnp.int32)
            .at[:N].set(neighbor_idx).reshape(NPAD * K))
    out = _sc_reduce(y2, posp, idxp)
    return out.reshape(NPAD, O)[:N]


# speedup vs baseline: 4.3039x; 1.2288x over previous
import functools

import jax
import jax.numpy as jnp
from jax import lax
from jax.experimental import pallas as pl
from jax.experimental.pallas import tpu as pltpu
from jax.experimental.pallas import tpu_sc as plsc

N, K, D, O = 10000, 32, 128, 128

NC, NS = 2, 16
NW = NC * NS
NPAD = 10240
RPW = NPAD // NW
CH = 4
NCHUNK = RPW // CH

MM_BM = 200


def _mm_body(h_ref, wt_ref, y_ref):
    y_ref[...] = jnp.dot(h_ref[...], wt_ref[...],
                         preferred_element_type=jnp.float32)


def _matmul(h, wt):
    return pl.pallas_call(
        _mm_body,
        grid=(N // MM_BM,),
        in_specs=[
            pl.BlockSpec((MM_BM, D), lambda i: (i, 0)),
            pl.BlockSpec((D, K * O), lambda i: (0, 0)),
        ],
        out_specs=pl.BlockSpec((MM_BM, K * O), lambda i: (i, 0)),
        out_shape=jax.ShapeDtypeStruct((N, K * O), jnp.float32),
    )(h, wt)


def _newton_rsqrt(sq):
    bits = plsc.bitcast(sq, jnp.int32)
    seed = jnp.int32(0x5F3759DF) - lax.shift_right_logical(bits, 1)
    y = plsc.bitcast(seed, jnp.float32)
    for _ in range(3):
        y = y * (jnp.float32(1.5) - jnp.float32(0.5) * sq * y * y)
    return jnp.where(sq == jnp.float32(0.0), jnp.float32(2.0), y)


def _sc_reduce(y2, posp, idxp):
    mesh = plsc.VectorSubcoreMesh(core_axis_name="c", subcore_axis_name="s")

    @functools.partial(
        pl.kernel,
        out_type=jax.ShapeDtypeStruct((NPAD * O,), jnp.float32),
        mesh=mesh,
        scratch_types=[
            pltpu.VMEM((NPAD + 16,), jnp.float32),
            pltpu.VMEM((NPAD + 16,), jnp.float32),
            pltpu.VMEM((NPAD + 16,), jnp.float32),
            pltpu.VMEM((RPW * K,), jnp.int32),
            pltpu.VMEM((CH * K,), jnp.int32),
            pltpu.VMEM((CH * K,), jnp.int32),
            pltpu.VMEM((CH * K, O), jnp.float32),
            pltpu.VMEM((CH * K, O), jnp.float32),
            pltpu.VMEM((CH * K + 16,), jnp.float32),
            pltpu.VMEM((CH * K + 16,), jnp.float32),
            pltpu.VMEM((CH * O,), jnp.float32),
            pltpu.SemaphoreType.DMA,
            pltpu.SemaphoreType.DMA,
        ],
        compiler_params=pltpu.CompilerParams(needs_layout_passes=False),
    )
    def sc_kernel(y_hbm, pos_hbm, idx_hbm, out_hbm,
                  posx, posy, posz, idxv, gidx0, gidx1, rows0, rows1,
                  inv0, inv1, outb, sem0, sem1):
        wid = lax.axis_index("s") * NC + lax.axis_index("c")
        base = wid * RPW
        pltpu.sync_copy(pos_hbm.at[pl.ds(0, NPAD)], posx.at[pl.ds(0, NPAD)])
        pltpu.sync_copy(pos_hbm.at[pl.ds(NPAD, NPAD)],
                        posy.at[pl.ds(0, NPAD)])
        pltpu.sync_copy(pos_hbm.at[pl.ds(2 * NPAD, NPAD)],
                        posz.at[pl.ds(0, NPAD)])
        pltpu.sync_copy(idx_hbm.at[pl.ds(base * K, RPW * K)], idxv)
        lane = lax.iota(jnp.int32, 16)

        def prepare(c, gidx, inv):
            for ii in range(CH):
                r = c * CH + ii
                g = base + r
                xi = posx[pl.ds(g, 16)][0]
                yi = posy[pl.ds(g, 16)][0]
                zi = posz[pl.ds(g, 16)][0]
                for half in range(2):
                    v = idxv[pl.ds(r * K + half * 16, 16)]
                    gidx[pl.ds(ii * K + half * 16, 16)] = (
                        v * K + (lane + half * 16))
                    dx = xi - plsc.load_gather(posx, [v])
                    dy = yi - plsc.load_gather(posy, [v])
                    dz = zi - plsc.load_gather(posz, [v])
                    sq = dx * dx + dy * dy + dz * dz
                    inv[pl.ds(ii * K + half * 16, 16)] = _newton_rsqrt(sq)

        def consume(c, rows, inv):
            for ii in range(CH):
                def acc_body(kk, acc):
                    j = ii * K + kk
                    s = inv[pl.ds(j, 16)][0]
                    return tuple(acc[u] + s * rows[j, pl.ds(u * 16, 16)]
                                 for u in range(8))
                acc = lax.fori_loop(
                    0, K, acc_body,
                    tuple(jnp.zeros((16,), jnp.float32) for _ in range(8)))
                for u in range(8):
                    outb[pl.ds(ii * O + u * 16, 16)] = acc[u]
            pltpu.sync_copy(
                outb, out_hbm.at[pl.ds((base + c * CH) * O, CH * O)])

        def start(gidx, rows, sem):
            pltpu.async_copy(y_hbm.at[gidx], rows, sem)

        def wait(gidx, rows, sem):
            pltpu.make_async_copy(y_hbm.at[gidx], rows, sem).wait()

        prepare(0, gidx0, inv0)
        start(gidx0, rows0, sem0)

        def step(t, carry):
            c0 = 2 * t
            prepare(c0 + 1, gidx1, inv1)
            start(gidx1, rows1, sem1)
            wait(gidx0, rows0, sem0)
            consume(c0, rows0, inv0)

            @pl.when(t < NCHUNK // 2 - 1)
            def _():
                prepare(c0 + 2, gidx0, inv0)
                start(gidx0, rows0, sem0)

            wait(gidx1, rows1, sem1)
            consume(c0 + 1, rows1, inv1)
            return carry

        lax.fori_loop(0, NCHUNK // 2, step, 0)

    return sc_kernel(y2, posp, idxp)


@jax.jit
def kernel(h, pos, neighbor_idx, W):
    wt = W.reshape(K, D, O).transpose(1, 0, 2).reshape(D, K * O)
    y2 = _matmul(h, wt).reshape(N * K, O)
    posp = (jnp.zeros((3, NPAD), jnp.float32)
            .at[:, :N].set(pos.T).reshape(3 * NPAD))
    idxp = (jnp.zeros((NPAD, K), jnp.int32)
            .at[:N].set(neighbor_idx).reshape(NPAD * K))
    out = _sc_reduce(y2, posp, idxp)
    return out.reshape(NPAD, O)[:N]


# speedup vs baseline: 4.3065x; 1.0006x over previous
import functools

import jax
import jax.numpy as jnp
from jax import lax
from jax.experimental import pallas as pl
from jax.experimental.pallas import tpu as pltpu
from jax.experimental.pallas import tpu_sc as plsc

N, K, D, O = 10000, 32, 128, 128

NC, NS = 2, 16
NW = NC * NS
NPAD = 10240
RPW = NPAD // NW
CH = 4
NCHUNK = RPW // CH

MM_BM = 200


def _mm_body(h_ref, wt_ref, y_ref):
    y_ref[...] = jnp.dot(h_ref[...], wt_ref[...],
                         preferred_element_type=jnp.float32)


def _matmul(h, wt):
    return pl.pallas_call(
        _mm_body,
        grid=(N // MM_BM,),
        in_specs=[
            pl.BlockSpec((MM_BM, D), lambda i: (i, 0)),
            pl.BlockSpec((D, K * O), lambda i: (0, 0)),
        ],
        out_specs=pl.BlockSpec((MM_BM, K * O), lambda i: (i, 0)),
        out_shape=jax.ShapeDtypeStruct((N, K * O), jnp.float32),
    )(h, wt)


def _newton_rsqrt(sq):
    bits = plsc.bitcast(sq, jnp.int32)
    seed = jnp.int32(0x5F3759DF) - lax.shift_right_logical(bits, 1)
    y = plsc.bitcast(seed, jnp.float32)
    for _ in range(3):
        y = y * (jnp.float32(1.5) - jnp.float32(0.5) * sq * y * y)
    return jnp.where(sq == jnp.float32(0.0), jnp.float32(2.0), y)


def _sc_reduce(y2, posp, idxp):
    mesh = plsc.VectorSubcoreMesh(core_axis_name="c", subcore_axis_name="s")

    @functools.partial(
        pl.kernel,
        out_type=jax.ShapeDtypeStruct((NPAD * O,), jnp.float32),
        mesh=mesh,
        scratch_types=[
            pltpu.VMEM((NPAD + 16,), jnp.float32),
            pltpu.VMEM((NPAD + 16,), jnp.float32),
            pltpu.VMEM((NPAD + 16,), jnp.float32),
            pltpu.VMEM((RPW * K,), jnp.int32),
            pltpu.VMEM((CH * K,), jnp.int32),
            pltpu.VMEM((CH * K,), jnp.int32),
            pltpu.VMEM((CH * K, O), jnp.float32),
            pltpu.VMEM((CH * K, O), jnp.float32),
            pltpu.VMEM((CH * K + 16,), jnp.float32),
            pltpu.VMEM((CH * K + 16,), jnp.float32),
            pltpu.VMEM((CH * O,), jnp.float32),
            pltpu.SemaphoreType.DMA,
            pltpu.SemaphoreType.DMA,
        ],
        compiler_params=pltpu.CompilerParams(needs_layout_passes=False),
    )
    def sc_kernel(y_hbm, pos_hbm, idx_hbm, out_hbm,
                  posx, posy, posz, idxv, gidx0, gidx1, rows0, rows1,
                  inv0, inv1, outb, sem0, sem1):
        wid = lax.axis_index("s") * NC + lax.axis_index("c")
        base = wid * RPW
        pltpu.sync_copy(pos_hbm.at[pl.ds(0, NPAD)], posx.at[pl.ds(0, NPAD)])
        pltpu.sync_copy(pos_hbm.at[pl.ds(NPAD, NPAD)],
                        posy.at[pl.ds(0, NPAD)])
        pltpu.sync_copy(pos_hbm.at[pl.ds(2 * NPAD, NPAD)],
                        posz.at[pl.ds(0, NPAD)])
        pltpu.sync_copy(idx_hbm.at[pl.ds(base * K, RPW * K)], idxv)
        lane = lax.iota(jnp.int32, 16)

        def prepare(c, gidx, inv):
            for ii in range(CH):
                r = c * CH + ii
                g = base + r
                xi = posx[pl.ds(g, 16)][0]
                yi = posy[pl.ds(g, 16)][0]
                zi = posz[pl.ds(g, 16)][0]
                for half in range(2):
                    v = idxv[pl.ds(r * K + half * 16, 16)]
                    gidx[pl.ds(ii * K + half * 16, 16)] = (
                        v * K + (lane + half * 16))
                    dx = xi - plsc.load_gather(posx, [v])
                    dy = yi - plsc.load_gather(posy, [v])
                    dz = zi - plsc.load_gather(posz, [v])
                    sq = dx * dx + dy * dy + dz * dz
                    inv[pl.ds(ii * K + half * 16, 16)] = _newton_rsqrt(sq)

        def consume(c, rows, inv):
            for ii in range(CH):
                def acc_body(kk, acc):
                    j = ii * K + kk
                    s = inv[pl.ds(j, 16)][0]
                    return tuple(acc[u] + s * rows[j, pl.ds(u * 16, 16)]
                                 for u in range(8))
                acc = lax.fori_loop(
                    0, K, acc_body,
                    tuple(jnp.zeros((16,), jnp.float32) for _ in range(8)))
                for u in range(8):
                    outb[pl.ds(ii * O + u * 16, 16)] = acc[u]
            pltpu.sync_copy(
                outb, out_hbm.at[pl.ds((base + c * CH) * O, CH * O)])

        def start(gidx, rows, sem):
            pltpu.async_copy(y_hbm.at[gidx], rows, sem)

        def wait(gidx, rows, sem):
            pltpu.make_async_copy(y_hbm.at[gidx], rows, sem).wait()

        prepare(0, gidx0, inv0)
        start(gidx0, rows0, sem0)

        def step(t, carry):
            c0 = 2 * t
            prepare(c0 + 1, gidx1, inv1)
            start(gidx1, rows1, sem1)
            wait(gidx0, rows0, sem0)
            consume(c0, rows0, inv0)

            @pl.when(t < NCHUNK // 2 - 1)
            def _():
                prepare(c0 + 2, gidx0, inv0)
                start(gidx0, rows0, sem0)

            wait(gidx1, rows1, sem1)
            consume(c0 + 1, rows1, inv1)
            return carry

        lax.fori_loop(0, NCHUNK // 2, step, 0)

    return sc_kernel(y2, posp, idxp)


@jax.jit
def kernel(h, pos, neighbor_idx, W):
    wt = W.reshape(K, D, O).transpose(1, 0, 2).reshape(D, K * O)
    y = _matmul(h, wt)
    y2 = y.reshape(N * K, O)
    posp = (jnp.zeros((3, NPAD), jnp.float32)
            .at[:, :N].set(pos.T).reshape(3 * NPAD))
    idxp = (jnp.zeros((NPAD, K), jnp.int32)
            .at[:N].set(neighbor_idx).reshape(NPAD * K))
    out = _sc_reduce(y2, posp, idxp)
    return out.reshape(NPAD, O)[:N]


# speedup vs baseline: 4.3756x; 1.0161x over previous
import functools

import jax
import jax.numpy as jnp
from jax import lax
from jax.experimental import pallas as pl
from jax.experimental.pallas import tpu as pltpu
from jax.experimental.pallas import tpu_sc as plsc

N, K, D, O = 10000, 32, 128, 128

NC, NS = 2, 16
NW = NC * NS
NPAD = 10240
RPW = NPAD // NW
CH = 8
NCHUNK = RPW // CH

MM_BM = 200


def _mm_body(h_ref, wt_ref, y_ref):
    y_ref[...] = jnp.dot(h_ref[...], wt_ref[...],
                         preferred_element_type=jnp.float32)


def _matmul(h, wt):
    return pl.pallas_call(
        _mm_body,
        grid=(N // MM_BM,),
        in_specs=[
            pl.BlockSpec((MM_BM, D), lambda i: (i, 0)),
            pl.BlockSpec((D, K * O), lambda i: (0, 0)),
        ],
        out_specs=pl.BlockSpec((MM_BM, K * O), lambda i: (i, 0)),
        out_shape=jax.ShapeDtypeStruct((N, K * O), jnp.float32),
    )(h, wt)


def _newton_rsqrt(sq):
    bits = plsc.bitcast(sq, jnp.int32)
    seed = jnp.int32(0x5F3759DF) - lax.shift_right_logical(bits, 1)
    y = plsc.bitcast(seed, jnp.float32)
    for _ in range(3):
        y = y * (jnp.float32(1.5) - jnp.float32(0.5) * sq * y * y)
    return jnp.where(sq == jnp.float32(0.0), jnp.float32(2.0), y)


def _sc_reduce(y2, posp, idxp):
    mesh = plsc.VectorSubcoreMesh(core_axis_name="c", subcore_axis_name="s")

    @functools.partial(
        pl.kernel,
        out_type=jax.ShapeDtypeStruct((NPAD * O,), jnp.float32),
        mesh=mesh,
        scratch_types=[
            pltpu.VMEM((NPAD + 16,), jnp.float32),
            pltpu.VMEM((NPAD + 16,), jnp.float32),
            pltpu.VMEM((NPAD + 16,), jnp.float32),
            pltpu.VMEM((RPW * K,), jnp.int32),
            pltpu.VMEM((CH * K // 2,), jnp.int32),
            pltpu.VMEM((CH * K // 2,), jnp.int32),
            pltpu.VMEM((CH * K // 2,), jnp.int32),
            pltpu.VMEM((CH * K // 2,), jnp.int32),
            pltpu.VMEM((CH * K, O), jnp.float32),
            pltpu.VMEM((CH * K, O), jnp.float32),
            pltpu.VMEM((CH * K + 16,), jnp.float32),
            pltpu.VMEM((CH * K + 16,), jnp.float32),
            pltpu.VMEM((CH * O,), jnp.float32),
            pltpu.SemaphoreType.DMA,
            pltpu.SemaphoreType.DMA,
        ],
        compiler_params=pltpu.CompilerParams(needs_layout_passes=False),
    )
    def sc_kernel(y_hbm, pos_hbm, idx_hbm, out_hbm,
                  posx, posy, posz, idxv, gidx0a, gidx0b, gidx1a, gidx1b,
                  rows0, rows1, inv0, inv1, outb, sem0, sem1):
        gidx0 = (gidx0a, gidx0b)
        gidx1 = (gidx1a, gidx1b)
        wid = lax.axis_index("s") * NC + lax.axis_index("c")
        base = wid * RPW
        pltpu.sync_copy(pos_hbm.at[pl.ds(0, NPAD)], posx.at[pl.ds(0, NPAD)])
        pltpu.sync_copy(pos_hbm.at[pl.ds(NPAD, NPAD)],
                        posy.at[pl.ds(0, NPAD)])
        pltpu.sync_copy(pos_hbm.at[pl.ds(2 * NPAD, NPAD)],
                        posz.at[pl.ds(0, NPAD)])
        pltpu.sync_copy(idx_hbm.at[pl.ds(base * K, RPW * K)], idxv)
        lane = lax.iota(jnp.int32, 16)

        def prepare(c, gidx, inv):
            for ii in range(CH):
                r = c * CH + ii
                g = base + r
                gi = gidx[ii // (CH // 2)]
                go = (ii % (CH // 2)) * K
                xi = posx[pl.ds(g, 16)][0]
                yi = posy[pl.ds(g, 16)][0]
                zi = posz[pl.ds(g, 16)][0]
                for half in range(2):
                    v = idxv[pl.ds(r * K + half * 16, 16)]
                    gi[pl.ds(go + half * 16, 16)] = (
                        v * K + (lane + half * 16))
                    dx = xi - plsc.load_gather(posx, [v])
                    dy = yi - plsc.load_gather(posy, [v])
                    dz = zi - plsc.load_gather(posz, [v])
                    sq = dx * dx + dy * dy + dz * dz
                    inv[pl.ds(ii * K + half * 16, 16)] = _newton_rsqrt(sq)

        def consume(c, rows, inv):
            for ii in range(CH):
                def acc_body(kk, acc):
                    j = ii * K + kk
                    s = inv[pl.ds(j, 16)][0]
                    return tuple(acc[u] + s * rows[j, pl.ds(u * 16, 16)]
                                 for u in range(8))
                acc = lax.fori_loop(
                    0, K, acc_body,
                    tuple(jnp.zeros((16,), jnp.float32) for _ in range(8)))
                for u in range(8):
                    outb[pl.ds(ii * O + u * 16, 16)] = acc[u]
            pltpu.sync_copy(
                outb, out_hbm.at[pl.ds((base + c * CH) * O, CH * O)])

        HK = CH * K // 2

        def start(gidx, rows, sem):
            pltpu.async_copy(y_hbm.at[gidx[0]], rows.at[pl.ds(0, HK)], sem)
            pltpu.async_copy(y_hbm.at[gidx[1]], rows.at[pl.ds(HK, HK)], sem)

        def wait(gidx, rows, sem):
            pltpu.make_async_copy(
                y_hbm.at[gidx[0]], rows.at[pl.ds(0, HK)], sem).wait()
            pltpu.make_async_copy(
                y_hbm.at[gidx[1]], rows.at[pl.ds(HK, HK)], sem).wait()

        prepare(0, gidx0, inv0)
        start(gidx0, rows0, sem0)

        def step(t, carry):
            c0 = 2 * t
            prepare(c0 + 1, gidx1, inv1)
            start(gidx1, rows1, sem1)
            wait(gidx0, rows0, sem0)
            consume(c0, rows0, inv0)

            @pl.when(t < NCHUNK // 2 - 1)
            def _():
                prepare(c0 + 2, gidx0, inv0)
                start(gidx0, rows0, sem0)

            wait(gidx1, rows1, sem1)
            consume(c0 + 1, rows1, inv1)
            return carry

        lax.fori_loop(0, NCHUNK // 2, step, 0)

    return sc_kernel(y2, posp, idxp)


@jax.jit
def kernel(h, pos, neighbor_idx, W):
    wt = W.reshape(K, D, O).transpose(1, 0, 2).reshape(D, K * O)
    y = _matmul(h, wt)
    y2 = y.reshape(N * K, O)
    posp = (jnp.zeros((3, NPAD), jnp.float32)
            .at[:, :N].set(pos.T).reshape(3 * NPAD))
    idxp = (jnp.zeros((NPAD, K), jnp.int32)
            .at[:N].set(neighbor_idx).reshape(NPAD * K))
    out = _sc_reduce(y2, posp, idxp)
    return out.reshape(NPAD, O)[:N]
